# baseline (speedup 1.0000x reference)
import functools

import jax
import jax.numpy as jnp
from jax import lax
from jax.experimental import pallas as pl
from jax.experimental.pallas import tpu as pltpu
from jax.experimental.pallas import tpu_sc as plsc

NC = 2
NS = 16
NW = NC * NS
L = 16
ROW = 32
P = 2048
B_SC = 6
HR = 128

_MESH = plsc.VectorSubcoreMesh(core_axis_name="c", subcore_axis_name="s")
_SC_PARAMS = pltpu.CompilerParams(needs_layout_passes=False)


def _argmax_tree(items):
    while len(items) > 1:
        nxt = []
        for j in range(0, len(items) - 1, 2):
            pm, pa = items[j]
            qm, qa = items[j + 1]
            gt = qm > pm
            nxt.append((jnp.where(gt, qm, pm), jnp.where(gt, qa, pa)))
        if len(items) % 2:
            nxt.append(items[-1])
        items = nxt
    return items[0]


def _sc_partial_hist(inp, tgt, C):
    _, _, H, W = inp.shape
    ppw = H * W // NW
    nchunk = ppw // P
    total = B_SC * nchunk
    RPC = P // W
    rpt = ppw // W

    @functools.partial(
        pl.kernel,
        mesh=_MESH,
        compiler_params=_SC_PARAMS,
        out_type=jax.ShapeDtypeStruct((NW, L, C * ROW), jnp.int32),
        scratch_types=[
            pltpu.VMEM((2, C, RPC, W), jnp.float32),
            pltpu.VMEM((2, RPC, W), jnp.int32),
            pltpu.VMEM((L, C * ROW), jnp.int32),
            pltpu.SemaphoreType.DMA,
            pltpu.SemaphoreType.DMA,
        ],
    )
    def k(in_hbm, tg_hbm, out_hbm, xbufs, tbufs, hist, sem0, sem1):
        wid = lax.axis_index("s") * NC + lax.axis_index("c")
        sems = (sem0, sem1)
        HB = C * ROW
        lane = lax.broadcasted_iota(jnp.int32, (L,), 0)
        zeros = jnp.zeros((L,), jnp.int32)
        ones = jnp.ones((L,), jnp.int32)

        def zero_body(i, _):
            for l in range(L):
                hist[l, pl.ds(i * L, L)] = zeros
            return 0

        lax.fori_loop(0, HB // L, zero_body, 0)

        def issue(ci, slot):
            b = ci // nchunk
            r0 = wid * rpt + (ci % nchunk) * RPC
            pltpu.async_copy(in_hbm.at[b, :, pl.ds(r0, RPC), :],
                             xbufs.at[slot], sems[slot])
            pltpu.async_copy(tg_hbm.at[b, pl.ds(r0, RPC), :],
                             tbufs.at[slot], sems[slot])

        def wait(slot):
            pltpu.make_async_copy(in_hbm.at[0, :, pl.ds(0, RPC), :],
                                  xbufs.at[slot], sems[slot]).wait()
            pltpu.make_async_copy(tg_hbm.at[0, pl.ds(0, RPC), :],
                                  tbufs.at[slot], sems[slot]).wait()

        def group(slot, r, col):
            _, a = _argmax_tree(
                [(xbufs[slot, c, r, pl.ds(col, L)], c) for c in range(C)])
            t = tbufs[slot, r, pl.ds(col, L)]
            plsc.addupdate_scatter(hist, [lane, t * ROW + a], ones)

        def compute(slot):
            for r in range(RPC):
                @plsc.parallel_loop(0, W // L, unroll=8)
                def _(i):
                    group(slot, r, i * L)

        issue(0, 0)
        issue(1, 1)

        def pair_body(cp, _):
            ci = cp * 2
            wait(0)
            compute(0)

            @pl.when(ci + 2 < total)
            def _():
                issue(ci + 2, 0)

            wait(1)
            compute(1)

            @pl.when(ci + 3 < total)
            def _():
                issue(ci + 3, 1)

            return 0

        lax.fori_loop(0, total // 2, pair_body, 0)
        pltpu.sync_copy(hist, out_hbm.at[wid])

    return k(inp, tgt)


def _tc_categories(inp, tgt, C):
    B, _, H, W = inp.shape
    bpb = H // HR
    nb = (B - B_SC) * bpb

    def body(x_ref, t_ref, o_ref):
        _, a = _argmax_tree([(x_ref[0, c], c) for c in range(C)])
        o_ref[0] = t_ref[0] * ROW + a

    return pl.pallas_call(
        body,
        grid=(nb,),
        in_specs=[
            pl.BlockSpec((1, C, HR, W),
                         lambda i: (B_SC + i // bpb, 0, i % bpb, 0)),
            pl.BlockSpec((1, HR, W),
                         lambda i: (B_SC + i // bpb, i % bpb, 0)),
        ],
        out_specs=pl.BlockSpec((1, HR, W), lambda i: (i // bpb, i % bpb, 0)),
        out_shape=jax.ShapeDtypeStruct((B - B_SC, H, W), jnp.int32),
    )(inp, tgt)


def _sc_cat_hist(cat, C):
    Bt, H, W = cat.shape
    ppw = H * W // NW
    nchunk = ppw // P
    total = Bt * nchunk
    RPC = P // W
    rpt = ppw // W

    @functools.partial(
        pl.kernel,
        mesh=_MESH,
        compiler_params=_SC_PARAMS,
        out_type=jax.ShapeDtypeStruct((NW, L, C * ROW), jnp.int32),
        scratch_types=[
            pltpu.VMEM((2, RPC, W), jnp.int32),
            pltpu.VMEM((L, C * ROW), jnp.int32),
            pltpu.SemaphoreType.DMA,
            pltpu.SemaphoreType.DMA,
        ],
    )
    def k(cat_hbm, out_hbm, cbufs, hist, sem0, sem1):
        wid = lax.axis_index("s") * NC + lax.axis_index("c")
        sems = (sem0, sem1)
        HB = C * ROW
        lane = lax.broadcasted_iota(jnp.int32, (L,), 0)
        zeros = jnp.zeros((L,), jnp.int32)
        ones = jnp.ones((L,), jnp.int32)

        def zero_body(i, _):
            for l in range(L):
                hist[l, pl.ds(i * L, L)] = zeros
            return 0

        lax.fori_loop(0, HB // L, zero_body, 0)

        def issue(ci, slot):
            b = ci // nchunk
            r0 = wid * rpt + (ci % nchunk) * RPC
            pltpu.async_copy(cat_hbm.at[b, pl.ds(r0, RPC), :],
                             cbufs.at[slot], sems[slot])

        def wait(slot):
            pltpu.make_async_copy(cat_hbm.at[0, pl.ds(0, RPC), :],
                                  cbufs.at[slot], sems[slot]).wait()

        def compute(slot):
            for r in range(RPC):
                @plsc.parallel_loop(0, W // L, unroll=8)
                def _(i):
                    catv = cbufs[slot, r, pl.ds(i * L, L)]
                    plsc.addupdate_scatter(hist, [lane, catv], ones)

        issue(0, 0)
        issue(1, 1)

        def pair_body(cp, _):
            ci = cp * 2
            wait(0)
            compute(0)

            @pl.when(ci + 2 < total)
            def _():
                issue(ci + 2, 0)

            wait(1)
            compute(1)

            @pl.when(ci + 3 < total)
            def _():
                issue(ci + 3, 1)

            return 0

        lax.fori_loop(0, total // 2, pair_body, 0)
        pltpu.sync_copy(hist, out_hbm.at[wid])

    return k(cat)


def _merge(parts_a, parts_b, C):
    def body(x_ref, y_ref, o_ref):
        o_ref[...] = (jnp.sum(x_ref[...], axis=(0, 1))
                      + jnp.sum(y_ref[...], axis=(0, 1)))

    return pl.pallas_call(
        body,
        out_shape=jax.ShapeDtypeStruct((C * ROW,), jnp.int32),
    )(parts_a, parts_b)


def kernel(input, target, class_num):
    C = input.shape[1]
    sc_parts = _sc_partial_hist(input, target, C)
    cat = _tc_categories(input, target, C)
    cat, sc_parts = lax.optimization_barrier((cat, sc_parts))
    cat_parts = _sc_cat_hist(cat, C)
    flat = _merge(sc_parts, cat_parts, C)
    return flat.reshape(C, ROW)[:, :C]


# speedup vs baseline: 1.2367x; 1.2367x over previous
import functools

import jax
import jax.numpy as jnp
from jax import lax
from jax.experimental import pallas as pl
from jax.experimental.pallas import tpu as pltpu
from jax.experimental.pallas import tpu_sc as plsc

NC = 2
NS = 16
NW = NC * NS
L = 16
ROW = 32
P = 2048
B_SC = 6
HR = 128

_MESH = plsc.VectorSubcoreMesh(core_axis_name="c", subcore_axis_name="s")
_SC_PARAMS = pltpu.CompilerParams(needs_layout_passes=False)


def _argmax_tree(items):
    while len(items) > 1:
        nxt = []
        for j in range(0, len(items) - 1, 2):
            pm, pa = items[j]
            qm, qa = items[j + 1]
            gt = qm > pm
            nxt.append((jnp.where(gt, qm, pm), jnp.where(gt, qa, pa)))
        if len(items) % 2:
            nxt.append(items[-1])
        items = nxt
    return items[0]


def _sc_partial_hist(inp, tgt, C):
    _, _, H, W = inp.shape
    ppw = H * W // NW
    nchunk = ppw // P
    total = B_SC * nchunk
    RPC = P // W
    rpt = ppw // W

    @functools.partial(
        pl.kernel,
        mesh=_MESH,
        compiler_params=_SC_PARAMS,
        out_type=jax.ShapeDtypeStruct((NW, L, C * ROW), jnp.int32),
        scratch_types=[
            pltpu.VMEM((2, C, RPC, W), jnp.float32),
            pltpu.VMEM((2, RPC, W), jnp.int32),
            pltpu.VMEM((L, C * ROW), jnp.int32),
            pltpu.SemaphoreType.DMA,
            pltpu.SemaphoreType.DMA,
        ],
    )
    def k(in_hbm, tg_hbm, out_hbm, xbufs, tbufs, hist, sem0, sem1):
        wid = lax.axis_index("s") * NC + lax.axis_index("c")
        sems = (sem0, sem1)
        HB = C * ROW
        lane = lax.broadcasted_iota(jnp.int32, (L,), 0)
        zeros = jnp.zeros((L,), jnp.int32)
        ones = jnp.ones((L,), jnp.int32)

        def zero_body(i, _):
            for l in range(L):
                hist[l, pl.ds(i * L, L)] = zeros
            return 0

        lax.fori_loop(0, HB // L, zero_body, 0)

        def issue(ci, slot):
            b = ci // nchunk
            r0 = wid * rpt + (ci % nchunk) * RPC
            pltpu.async_copy(in_hbm.at[b, :, pl.ds(r0, RPC), :],
                             xbufs.at[slot], sems[slot])
            pltpu.async_copy(tg_hbm.at[b, pl.ds(r0, RPC), :],
                             tbufs.at[slot], sems[slot])

        def wait(slot):
            pltpu.make_async_copy(in_hbm.at[0, :, pl.ds(0, RPC), :],
                                  xbufs.at[slot], sems[slot]).wait()
            pltpu.make_async_copy(tg_hbm.at[0, pl.ds(0, RPC), :],
                                  tbufs.at[slot], sems[slot]).wait()

        def group(slot, r, col):
            _, a = _argmax_tree(
                [(xbufs[slot, c, r, pl.ds(col, L)], c) for c in range(C)])
            t = tbufs[slot, r, pl.ds(col, L)]
            plsc.addupdate_scatter(hist, [lane, t * ROW + a], ones)

        def compute(slot):
            for r in range(RPC):
                @plsc.parallel_loop(0, W // L, unroll=4)
                def _(i):
                    group(slot, r, i * L)

        issue(0, 0)
        issue(1, 1)

        def pair_body(cp, _):
            ci = cp * 2
            wait(0)
            compute(0)

            @pl.when(ci + 2 < total)
            def _():
                issue(ci + 2, 0)

            wait(1)
            compute(1)

            @pl.when(ci + 3 < total)
            def _():
                issue(ci + 3, 1)

            return 0

        lax.fori_loop(0, total // 2, pair_body, 0)
        pltpu.sync_copy(hist, out_hbm.at[wid])

    return k(inp, tgt)


def _tc_categories(inp, tgt, C):
    B, _, H, W = inp.shape
    bpb = H // HR
    nb = (B - B_SC) * bpb

    def body(x_ref, t_ref, o_ref):
        _, a = _argmax_tree([(x_ref[0, c], c) for c in range(C)])
        o_ref[0] = t_ref[0] * ROW + a

    return pl.pallas_call(
        body,
        grid=(nb,),
        in_specs=[
            pl.BlockSpec((1, C, HR, W),
                         lambda i: (B_SC + i // bpb, 0, i % bpb, 0)),
            pl.BlockSpec((1, HR, W),
                         lambda i: (B_SC + i // bpb, i % bpb, 0)),
        ],
        out_specs=pl.BlockSpec((1, HR, W), lambda i: (i // bpb, i % bpb, 0)),
        out_shape=jax.ShapeDtypeStruct((B - B_SC, H, W), jnp.int32),
    )(inp, tgt)


def _sc_cat_hist(cat, C):
    Bt, H, W = cat.shape
    ppw = H * W // NW
    nchunk = ppw // P
    total = Bt * nchunk
    RPC = P // W
    rpt = ppw // W

    @functools.partial(
        pl.kernel,
        mesh=_MESH,
        compiler_params=_SC_PARAMS,
        out_type=jax.ShapeDtypeStruct((NW, L, C * ROW), jnp.int32),
        scratch_types=[
            pltpu.VMEM((2, RPC, W), jnp.int32),
            pltpu.VMEM((L, C * ROW), jnp.int32),
            pltpu.SemaphoreType.DMA,
            pltpu.SemaphoreType.DMA,
        ],
    )
    def k(cat_hbm, out_hbm, cbufs, hist, sem0, sem1):
        wid = lax.axis_index("s") * NC + lax.axis_index("c")
        sems = (sem0, sem1)
        HB = C * ROW
        lane = lax.broadcasted_iota(jnp.int32, (L,), 0)
        zeros = jnp.zeros((L,), jnp.int32)
        ones = jnp.ones((L,), jnp.int32)

        def zero_body(i, _):
            for l in range(L):
                hist[l, pl.ds(i * L, L)] = zeros
            return 0

        lax.fori_loop(0, HB // L, zero_body, 0)

        def issue(ci, slot):
            b = ci // nchunk
            r0 = wid * rpt + (ci % nchunk) * RPC
            pltpu.async_copy(cat_hbm.at[b, pl.ds(r0, RPC), :],
                             cbufs.at[slot], sems[slot])

        def wait(slot):
            pltpu.make_async_copy(cat_hbm.at[0, pl.ds(0, RPC), :],
                                  cbufs.at[slot], sems[slot]).wait()

        def compute(slot):
            for r in range(RPC):
                @plsc.parallel_loop(0, W // L, unroll=4)
                def _(i):
                    catv = cbufs[slot, r, pl.ds(i * L, L)]
                    plsc.addupdate_scatter(hist, [lane, catv], ones)

        issue(0, 0)
        issue(1, 1)

        def pair_body(cp, _):
            ci = cp * 2
            wait(0)
            compute(0)

            @pl.when(ci + 2 < total)
            def _():
                issue(ci + 2, 0)

            wait(1)
            compute(1)

            @pl.when(ci + 3 < total)
            def _():
                issue(ci + 3, 1)

            return 0

        lax.fori_loop(0, total // 2, pair_body, 0)
        pltpu.sync_copy(hist, out_hbm.at[wid])

    return k(cat)


def _merge(parts_a, parts_b, C):
    def body(x_ref, y_ref, o_ref):
        o_ref[...] = (jnp.sum(x_ref[...], axis=(0, 1))
                      + jnp.sum(y_ref[...], axis=(0, 1)))

    return pl.pallas_call(
        body,
        out_shape=jax.ShapeDtypeStruct((C * ROW,), jnp.int32),
    )(parts_a, parts_b)


def kernel(input, target, class_num):
    C = input.shape[1]
    sc_parts = _sc_partial_hist(input, target, C)
    cat = _tc_categories(input, target, C)
    cat, sc_parts = lax.optimization_barrier((cat, sc_parts))
    cat_parts = _sc_cat_hist(cat, C)
    flat = _merge(sc_parts, cat_parts, C)
    return flat.reshape(C, ROW)[:, :C]


# speedup vs baseline: 1.2408x; 1.0033x over previous
import functools

import jax
import jax.numpy as jnp
from jax import lax
from jax.experimental import pallas as pl
from jax.experimental.pallas import tpu as pltpu
from jax.experimental.pallas import tpu_sc as plsc

NC = 2
NS = 16
NW = NC * NS
L = 16
ROW = 32
P = 2048
B_SC = 6
HR = 256

_MESH = plsc.VectorSubcoreMesh(core_axis_name="c", subcore_axis_name="s")
_SC_PARAMS = pltpu.CompilerParams(needs_layout_passes=False)


def _argmax_tree(items):
    while len(items) > 1:
        nxt = []
        for j in range(0, len(items) - 1, 2):
            pm, pa = items[j]
            qm, qa = items[j + 1]
            gt = qm > pm
            nxt.append((jnp.where(gt, qm, pm), jnp.where(gt, qa, pa)))
        if len(items) % 2:
            nxt.append(items[-1])
        items = nxt
    return items[0]


def _sc_partial_hist(inp, tgt, C):
    _, _, H, W = inp.shape
    ppw = H * W // NW
    nchunk = ppw // P
    total = B_SC * nchunk
    RPC = P // W
    rpt = ppw // W

    @functools.partial(
        pl.kernel,
        mesh=_MESH,
        compiler_params=_SC_PARAMS,
        out_type=jax.ShapeDtypeStruct((NW, L, C * ROW), jnp.int32),
        scratch_types=[
            pltpu.VMEM((2, C, RPC, W), jnp.float32),
            pltpu.VMEM((2, RPC, W), jnp.int32),
            pltpu.VMEM((L, C * ROW), jnp.int32),
            pltpu.SemaphoreType.DMA,
            pltpu.SemaphoreType.DMA,
        ],
    )
    def k(in_hbm, tg_hbm, out_hbm, xbufs, tbufs, hist, sem0, sem1):
        wid = lax.axis_index("s") * NC + lax.axis_index("c")
        sems = (sem0, sem1)
        HB = C * ROW
        lane = lax.broadcasted_iota(jnp.int32, (L,), 0)
        zeros = jnp.zeros((L,), jnp.int32)
        ones = jnp.ones((L,), jnp.int32)

        def zero_body(i, _):
            for l in range(L):
                hist[l, pl.ds(i * L, L)] = zeros
            return 0

        lax.fori_loop(0, HB // L, zero_body, 0)

        def issue(ci, slot):
            b = ci // nchunk
            r0 = wid * rpt + (ci % nchunk) * RPC
            pltpu.async_copy(in_hbm.at[b, :, pl.ds(r0, RPC), :],
                             xbufs.at[slot], sems[slot])
            pltpu.async_copy(tg_hbm.at[b, pl.ds(r0, RPC), :],
                             tbufs.at[slot], sems[slot])

        def wait(slot):
            pltpu.make_async_copy(in_hbm.at[0, :, pl.ds(0, RPC), :],
                                  xbufs.at[slot], sems[slot]).wait()
            pltpu.make_async_copy(tg_hbm.at[0, pl.ds(0, RPC), :],
                                  tbufs.at[slot], sems[slot]).wait()

        def group(slot, r, col):
            _, a = _argmax_tree(
                [(xbufs[slot, c, r, pl.ds(col, L)], c) for c in range(C)])
            t = tbufs[slot, r, pl.ds(col, L)]
            plsc.addupdate_scatter(hist, [lane, t * ROW + a], ones)

        def compute(slot):
            for r in range(RPC):
                @plsc.parallel_loop(0, W // L, unroll=4)
                def _(i):
                    group(slot, r, i * L)

        issue(0, 0)
        issue(1, 1)

        def pair_body(cp, _):
            ci = cp * 2
            wait(0)
            compute(0)

            @pl.when(ci + 2 < total)
            def _():
                issue(ci + 2, 0)

            wait(1)
            compute(1)

            @pl.when(ci + 3 < total)
            def _():
                issue(ci + 3, 1)

            return 0

        lax.fori_loop(0, total // 2, pair_body, 0)
        pltpu.sync_copy(hist, out_hbm.at[wid])

    return k(inp, tgt)


def _tc_categories(inp, tgt, C):
    B, _, H, W = inp.shape
    bpb = H // HR
    nb = (B - B_SC) * bpb

    def body(x_ref, t_ref, o_ref):
        _, a = _argmax_tree([(x_ref[0, c], c) for c in range(C)])
        o_ref[0] = t_ref[0] * ROW + a

    return pl.pallas_call(
        body,
        grid=(nb,),
        in_specs=[
            pl.BlockSpec((1, C, HR, W),
                         lambda i: (B_SC + i // bpb, 0, i % bpb, 0)),
            pl.BlockSpec((1, HR, W),
                         lambda i: (B_SC + i // bpb, i % bpb, 0)),
        ],
        out_specs=pl.BlockSpec((1, HR, W), lambda i: (i // bpb, i % bpb, 0)),
        out_shape=jax.ShapeDtypeStruct((B - B_SC, H, W), jnp.int32),
    )(inp, tgt)


def _sc_cat_hist(cat, C):
    Bt, H, W = cat.shape
    ppw = H * W // NW
    nchunk = ppw // P
    total = Bt * nchunk
    RPC = P // W
    rpt = ppw // W

    @functools.partial(
        pl.kernel,
        mesh=_MESH,
        compiler_params=_SC_PARAMS,
        out_type=jax.ShapeDtypeStruct((NW, L, C * ROW), jnp.int32),
        scratch_types=[
            pltpu.VMEM((2, RPC, W), jnp.int32),
            pltpu.VMEM((L, C * ROW), jnp.int32),
            pltpu.SemaphoreType.DMA,
            pltpu.SemaphoreType.DMA,
        ],
    )
    def k(cat_hbm, out_hbm, cbufs, hist, sem0, sem1):
        wid = lax.axis_index("s") * NC + lax.axis_index("c")
        sems = (sem0, sem1)
        HB = C * ROW
        lane = lax.broadcasted_iota(jnp.int32, (L,), 0)
        zeros = jnp.zeros((L,), jnp.int32)
        ones = jnp.ones((L,), jnp.int32)

        def zero_body(i, _):
            for l in range(L):
                hist[l, pl.ds(i * L, L)] = zeros
            return 0

        lax.fori_loop(0, HB // L, zero_body, 0)

        def issue(ci, slot):
            b = ci // nchunk
            r0 = wid * rpt + (ci % nchunk) * RPC
            pltpu.async_copy(cat_hbm.at[b, pl.ds(r0, RPC), :],
                             cbufs.at[slot], sems[slot])

        def wait(slot):
            pltpu.make_async_copy(cat_hbm.at[0, pl.ds(0, RPC), :],
                                  cbufs.at[slot], sems[slot]).wait()

        def compute(slot):
            for r in range(RPC):
                @plsc.parallel_loop(0, W // L, unroll=4)
                def _(i):
                    catv = cbufs[slot, r, pl.ds(i * L, L)]
                    plsc.addupdate_scatter(hist, [lane, catv], ones)

        issue(0, 0)
        issue(1, 1)

        def pair_body(cp, _):
            ci = cp * 2
            wait(0)
            compute(0)

            @pl.when(ci + 2 < total)
            def _():
                issue(ci + 2, 0)

            wait(1)
            compute(1)

            @pl.when(ci + 3 < total)
            def _():
                issue(ci + 3, 1)

            return 0

        lax.fori_loop(0, total // 2, pair_body, 0)
        pltpu.sync_copy(hist, out_hbm.at[wid])

    return k(cat)


def _merge(parts_a, parts_b, C):
    def body(x_ref, y_ref, o_ref):
        o_ref[...] = (jnp.sum(x_ref[...], axis=(0, 1))
                      + jnp.sum(y_ref[...], axis=(0, 1)))

    return pl.pallas_call(
        body,
        out_shape=jax.ShapeDtypeStruct((C * ROW,), jnp.int32),
    )(parts_a, parts_b)


def kernel(input, target, class_num):
    C = input.shape[1]
    sc_parts = _sc_partial_hist(input, target, C)
    cat = _tc_categories(input, target, C)
    cat, sc_parts = lax.optimization_barrier((cat, sc_parts))
    cat_parts = _sc_cat_hist(cat, C)
    flat = _merge(sc_parts, cat_parts, C)
    return flat.reshape(C, ROW)[:, :C]
